# Initial kernel scaffold; baseline (speedup 1.0000x reference)
import jax
import jax.numpy as jnp
from jax.experimental import pallas as pl


def kernel(users, pos_items, neg_items, adj_rows, adj_cols, adj_vals, feature, w1, w2, w3, user_emb, item_emb1, item_emb2, W_gc_0, b_gc_0, W_bi_0, b_bi_0, W_gc_1, b_gc_1, W_bi_1, b_bi_1):
    raise NotImplementedError("write your pallas kernel here")


# speedup vs baseline: 4.8642x; 4.8642x over previous
import functools

import jax
import jax.numpy as jnp
from jax import lax
from jax.experimental import pallas as pl
from jax.experimental.pallas import tpu as pltpu
from jax.experimental.pallas import tpu_sc as plsc

N_USER = 25000
N_TOT = 50000
NNZ = 800000
BATCH = 4096
HALF = 32
NC, NS, L = 2, 16, 16

CH = 128
E_PAD = 819200
EPT = E_PAD // NS
NCH = EPT // CH
ACC_ROWS = 51200
STRIPE = ACC_ROWS // NS
G = 3 * BATCH
GPT = G // (NC * NS)

_MESH = dict(core_axis_name="c", subcore_axis_name="s", num_cores=NC,
             num_subcores=NS)


@functools.lru_cache(maxsize=None)
def _build_spmm():
  return functools.partial(
      pl.kernel,
      out_type=jax.ShapeDtypeStruct((2 * ACC_ROWS, HALF), jnp.float32),
      mesh=plsc.VectorSubcoreMesh(**_MESH),
      scratch_types=[
          pltpu.VMEM((CH,), jnp.int32),
          pltpu.VMEM((CH,), jnp.int32),
          pltpu.VMEM((CH,), jnp.float32),
          pltpu.VMEM((CH, HALF), jnp.float32),
          pltpu.VMEM((L,), jnp.float32),
          pltpu.VMEM_SHARED((ACC_ROWS, HALF), jnp.float32),
          pltpu.SemaphoreType.DMA,
      ],
      compiler_params=pltpu.CompilerParams(use_tc_tiling_on_sc=False),
  )(_spmm_body)


def _spmm_body(ego, rows_r, cols_r, vals_r, wv_r, out_r,
               colv, rowv, valv, gbuf, wvv, acc, sem):
    c = lax.axis_index("c")
    s = lax.axis_index("s")
    pltpu.sync_copy(wv_r, wvv)
    wv = wvv[pl.ds(0, L)]
    w1 = wv[0]
    w2 = wv[1]
    w3 = wv[2]

    z = jnp.zeros((L,), jnp.float32)

    def zrow(j, _):
        gbuf[j, pl.ds(0, L)] = z
        gbuf[j, pl.ds(L, L)] = z
        return 0

    lax.fori_loop(0, CH, zrow, 0)

    def zstripe(i, _):
        pltpu.sync_copy(gbuf, acc.at[pl.ds(s * STRIPE + i * CH, CH)])
        return 0

    lax.fori_loop(0, STRIPE // CH, zstripe, 0)
    plsc.subcore_barrier()

    cofs = c * N_TOT

    def chunk(i, _):
        base = s * EPT + i * CH
        pltpu.sync_copy(cols_r.at[pl.ds(base, CH)], colv)
        pltpu.sync_copy(rows_r.at[pl.ds(base, CH)], rowv)
        pltpu.sync_copy(vals_r.at[pl.ds(base, CH)], valv)
        for q in range(CH // L):
            sl = pl.ds(q * L, L)
            colv[sl] = colv[sl] + cofs
            v = valv[sl]
            v2 = v * v
            v6 = v2 * v2 * v2
            valv[sl] = v * w1 + v2 * w2 + v6 * w3
        pltpu.async_copy(ego.at[colv], gbuf, sem).wait()

        def scale(g, _):
            vvec = valv[pl.ds(g * L, L)]
            for jj in range(L):
                j = g * L + jj
                vv = vvec[jj]
                gbuf[j, pl.ds(0, L)] = gbuf[j, pl.ds(0, L)] * vv
                gbuf[j, pl.ds(L, L)] = gbuf[j, pl.ds(L, L)] * vv
            return 0

        lax.fori_loop(0, CH // L, scale, 0)
        pltpu.sync_copy(gbuf, acc.at[rowv], add=True)
        return 0

    lax.fori_loop(0, NCH, chunk, 0)
    plsc.subcore_barrier()

    def wb(i, _):
        o = s * STRIPE + i * CH
        pltpu.sync_copy(acc.at[pl.ds(o, CH)],
                        out_r.at[pl.ds(c * ACC_ROWS + o, CH)])
        return 0

    lax.fori_loop(0, STRIPE // CH, wb, 0)


@functools.lru_cache(maxsize=None)
def _build_gather():
  return functools.partial(
      pl.kernel,
      out_type=(jax.ShapeDtypeStruct((G, HALF), jnp.float32),
                jax.ShapeDtypeStruct((G, HALF), jnp.float32),
                jax.ShapeDtypeStruct((G, 2 * HALF), jnp.float32),
                jax.ShapeDtypeStruct((G, 2 * HALF), jnp.float32)),
      mesh=plsc.VectorSubcoreMesh(**_MESH),
      scratch_types=[
          pltpu.VMEM((CH,), jnp.int32),
          pltpu.VMEM((CH, 2 * HALF), jnp.float32),
          pltpu.VMEM((CH, HALF), jnp.float32),
          pltpu.SemaphoreType.DMA,
      ],
      compiler_params=pltpu.CompilerParams(use_tc_tiling_on_sc=False),
  )(_gather_body)


def _gather_body(ego0, n1, n2, sel_r, g0lo, g0hi, g1, g2,
                 idxv, buf64, buf32, sem):
    c = lax.axis_index("c")
    s = lax.axis_index("s")
    wid = s * NC + c
    for k in range(GPT // CH):
        base = wid * GPT + k * CH
        pltpu.sync_copy(sel_r.at[pl.ds(base, CH)], idxv)
        pltpu.async_copy(n1.at[idxv], buf64, sem).wait()
        pltpu.sync_copy(buf64, g1.at[pl.ds(base, CH)])
        pltpu.async_copy(n2.at[idxv], buf64, sem).wait()
        pltpu.sync_copy(buf64, g2.at[pl.ds(base, CH)])
        pltpu.async_copy(ego0.at[idxv], buf32, sem).wait()
        pltpu.sync_copy(buf32, g0lo.at[pl.ds(base, CH)])
        for q in range(CH // L):
            sl = pl.ds(q * L, L)
            idxv[sl] = idxv[sl] + N_TOT
        pltpu.async_copy(ego0.at[idxv], buf32, sem).wait()
        pltpu.sync_copy(buf32, g0hi.at[pl.ds(base, CH)])


BLK = 2000


def _dense_math(slo, shi, elo, ehi, wg, bg, wb, bb):
    side = jnp.concatenate([slo, shi], axis=1)
    ego = jnp.concatenate([elo, ehi], axis=1)
    se = jnp.dot(side, wg, preferred_element_type=jnp.float32) + bg
    bi = jnp.dot(ego * side, wb, preferred_element_type=jnp.float32) + bb
    x = se + bi
    e2 = jnp.where(x >= 0, x, 0.2 * x)
    nrm = jnp.sqrt(jnp.sum(e2 * e2, axis=1, keepdims=True))
    return e2, e2 / jnp.maximum(nrm, 1e-12)


def _dense_body_ego(slo, shi, elo, ehi, wg, bg, wb, bb, eout, nout):
    e2, nval = _dense_math(slo[...], shi[...], elo[...], ehi[...],
                           wg[...], bg[...], wb[...], bb[...])
    eout[0] = e2[:, :HALF]
    eout[1] = e2[:, HALF:]
    nout[...] = nval


def _dense_body_n(slo, shi, elo, ehi, wg, bg, wb, bb, nout):
    _, nval = _dense_math(slo[...], shi[...], elo[...], ehi[...],
                          wg[...], bg[...], wb[...], bb[...])
    nout[...] = nval


def _dense_call(slo, shi, elo, ehi, wg, bg, wb, bb, want_ego):
    row_spec = pl.BlockSpec((BLK, HALF), lambda i: (i, 0))
    w_spec = pl.BlockSpec((64, 64), lambda i: (0, 0))
    b_spec = pl.BlockSpec((1, 64), lambda i: (0, 0))
    in_specs = [row_spec, row_spec, row_spec, row_spec,
                w_spec, b_spec, w_spec, b_spec]
    n_shape = jax.ShapeDtypeStruct((N_TOT, 2 * HALF), jnp.float32)
    n_spec = pl.BlockSpec((BLK, 2 * HALF), lambda i: (i, 0))
    if want_ego:
        out_shape = (jax.ShapeDtypeStruct((2, N_TOT, HALF), jnp.float32),
                     n_shape)
        out_specs = (pl.BlockSpec((2, BLK, HALF), lambda i: (0, i, 0)),
                     n_spec)
        body = _dense_body_ego
    else:
        out_shape = n_shape
        out_specs = n_spec
        body = _dense_body_n
    return pl.pallas_call(
        body,
        grid=(N_TOT // BLK,),
        in_specs=in_specs,
        out_specs=out_specs,
        out_shape=out_shape,
    )(slo, shi, elo, ehi, wg, bg, wb, bb)


def kernel(users, pos_items, neg_items, adj_rows, adj_cols, adj_vals,
           feature, w1, w2, w3, user_emb, item_emb1, item_emb2,
           W_gc_0, b_gc_0, W_bi_0, b_bi_0, W_gc_1, b_gc_1, W_bi_1, b_bi_1):
    rows = adj_rows.astype(jnp.int32)
    cols = adj_cols.astype(jnp.int32)
    vals = adj_vals.astype(jnp.float32)
    pad = E_PAD - NNZ
    rows = jnp.concatenate([rows, jnp.zeros((pad,), jnp.int32)])
    cols = jnp.concatenate([cols, jnp.zeros((pad,), jnp.int32)])
    vals = jnp.concatenate([vals, jnp.zeros((pad,), jnp.float32)])
    wvec = jnp.concatenate(
        [w1, w2, w3, jnp.zeros((L - 3,), jnp.float32)]).astype(jnp.float32)

    ego_cat0 = jnp.concatenate([user_emb, item_emb1, feature, item_emb2],
                               axis=0)

    side1 = _build_spmm()(ego_cat0, rows, cols, vals, wvec)
    s1lo = side1[:N_TOT]
    s1hi = side1[ACC_ROWS:ACC_ROWS + N_TOT]
    e1pair, n1 = _dense_call(s1lo, s1hi, ego_cat0[:N_TOT], ego_cat0[N_TOT:],
                             W_gc_0, b_gc_0, W_bi_0, b_bi_0, True)
    ego_cat1 = e1pair.reshape(2 * N_TOT, HALF)

    side2 = _build_spmm()(ego_cat1, rows, cols, vals, wvec)
    s2lo = side2[:N_TOT]
    s2hi = side2[ACC_ROWS:ACC_ROWS + N_TOT]
    n2 = _dense_call(s2lo, s2hi, ego_cat1[:N_TOT], ego_cat1[N_TOT:],
                     W_gc_1, b_gc_1, W_bi_1, b_bi_1, False)

    sel = jnp.concatenate([users, pos_items + N_USER,
                           neg_items + N_USER]).astype(jnp.int32)
    g0lo, g0hi, g1, g2 = _build_gather()(ego_cat0, n1, n2, sel)

    def asm(a, b):
        return jnp.concatenate([g0lo[a:b], g0hi[a:b], g1[a:b], g2[a:b]],
                               axis=1)

    return (asm(0, BATCH), asm(BATCH, 2 * BATCH), asm(2 * BATCH, 3 * BATCH))
